# baseline (speedup 1.0000x reference)
import functools

import jax
import jax.numpy as jnp
from jax import lax
from jax.experimental import pallas as pl
from jax.experimental.pallas import tpu as pltpu
from jax.experimental.pallas import tpu_sc as plsc

K = 8
NEG = float("-inf")
IMAX = 2**31 - 1


def _extract_topk(scores, ncol, vars_):
    s_list, c_list, v_list = [], [], []
    work = scores
    for _ in range(K):
        m = jnp.max(work, axis=1, keepdims=True)
        hit = work == m
        nc = jnp.max(jnp.where(hit, ncol, NEG), axis=1, keepdims=True)
        sel = ncol == nc
        v = jnp.max(jnp.where(sel, vars_, NEG), axis=1, keepdims=True)
        work = jnp.where(sel, NEG, work)
        s_list.append(m)
        c_list.append(nc)
        v_list.append(v)
    return (jnp.concatenate(s_list, axis=1),
            jnp.concatenate(c_list, axis=1),
            jnp.concatenate(v_list, axis=1))


def _sim_topk_body(q_ref, emb_ref, w1_ref, b1_ref, w2_ref, b2_ref, mask_ref,
                   cand_out, *, chunk, nchunks):
    i = pl.program_id(0)
    b = q_ref.shape[0]
    mc = mask_ref.shape[0]

    q = q_ref[...]
    qn = q / jnp.maximum(jnp.sqrt(jnp.sum(q * q, axis=-1, keepdims=True)),
                         1e-8)
    emb = emb_ref[...]
    h = jnp.maximum(jnp.dot(emb, w1_ref[...]) + b1_ref[...], 0.0)
    ssum = None
    ssq = None
    for m_i in range(mc):
        noisy = jnp.dot(h * mask_ref[m_i], w2_ref[...]) + b2_ref[...]
        en = noisy / jnp.maximum(
            jnp.sqrt(jnp.sum(noisy * noisy, axis=-1, keepdims=True)), 1e-8)
        s = lax.dot_general(qn, en, (((1,), (1,)), ((), ())))
        ssum = s if ssum is None else ssum + s
        ssq = s * s if ssq is None else ssq + s * s
    mean = ssum / mc
    var = (ssq - mc * (mean * mean)) / (mc - 1)

    ncol = (-(lax.broadcasted_iota(jnp.int32, (b, chunk), 1)
              + i * chunk)).astype(jnp.float32)
    ch_s, ch_c, ch_v = _extract_topk(mean, ncol, var)

    cand_out[0] = jnp.concatenate([ch_s, ch_c, ch_v], axis=1)


def _sim_topk(query, emb, w1, b1, w2, b2, masks, *, chunk, interpret=False):
    b, h_dim = query.shape
    cap = emb.shape[0]
    mc = masks.shape[0]
    assert cap % chunk == 0
    nchunks = cap // chunk
    body = functools.partial(_sim_topk_body, chunk=chunk, nchunks=nchunks)
    return pl.pallas_call(
        body,
        grid=(nchunks,),
        in_specs=[
            pl.BlockSpec((b, h_dim), lambda i: (0, 0)),
            pl.BlockSpec((chunk, h_dim), lambda i: (i, 0)),
            pl.BlockSpec((h_dim, h_dim), lambda i: (0, 0)),
            pl.BlockSpec((1, h_dim), lambda i: (0, 0)),
            pl.BlockSpec((h_dim, h_dim), lambda i: (0, 0)),
            pl.BlockSpec((1, h_dim), lambda i: (0, 0)),
            pl.BlockSpec((mc, chunk, h_dim), lambda i: (0, i, 0)),
        ],
        out_specs=pl.BlockSpec((1, b, 3 * K), lambda i: (i, 0, 0)),
        out_shape=jax.ShapeDtypeStruct((nchunks, b, 3 * K), jnp.float32),
        interpret=interpret,
    )(query, emb, w1, b1, w2, b2, masks)


def _final_topk_body(s_ref, c_ref, v_ref, s_out, i_out, v_out):
    f_s, f_c, f_v = _extract_topk(s_ref[...], c_ref[...], v_ref[...])
    s_out[...] = f_s
    i_out[...] = (-f_c).astype(jnp.int32)
    v_out[...] = f_v


def _final_topk(cand_s, cand_c, cand_v, *, interpret=False):
    b = cand_s.shape[0]
    return pl.pallas_call(
        _final_topk_body,
        out_shape=[
            jax.ShapeDtypeStruct((b, K), jnp.float32),
            jax.ShapeDtypeStruct((b, K), jnp.int32),
            jax.ShapeDtypeStruct((b, K), jnp.float32),
        ],
        interpret=interpret,
    )(cand_s, cand_c, cand_v)


def _sc_gather(table, idx):
    n = idx.shape[0]
    _, seq, h_dim = table.shape
    info = plsc.get_sparse_core_info()
    nw = info.num_cores * info.num_subcores
    bpw = n // nw
    ch = 32
    assert bpw % ch == 0
    nch = bpw // ch
    mesh = plsc.VectorSubcoreMesh(core_axis_name="c", subcore_axis_name="s")

    @functools.partial(
        pl.kernel,
        out_type=jax.ShapeDtypeStruct((n, seq, h_dim), jnp.float32),
        mesh=mesh,
        scratch_types=[
            pltpu.VMEM((ch,), jnp.int32),
            pltpu.VMEM((ch, seq, h_dim), jnp.float32),
            pltpu.SemaphoreType.DMA,
        ],
    )
    def gather_kernel(table_hbm, idx_hbm, out_hbm, idx_v, rows_v, sem):
        wid = lax.axis_index("s") * info.num_cores + lax.axis_index("c")
        base = wid * bpw

        def step(c, carry):
            off = base + c * ch
            pltpu.sync_copy(idx_hbm.at[pl.ds(off, ch)], idx_v)
            pltpu.async_copy(table_hbm.at[idx_v], rows_v, sem).wait()
            pltpu.sync_copy(rows_v, out_hbm.at[pl.ds(off, ch)])
            return carry

        lax.fori_loop(0, nch, step, 0)

    return gather_kernel(table, idx)


def kernel(query, k, episodes, episode_embeddings, W1, b1, W2, b2, drop_masks):
    del k
    cap, seq, h_dim = episodes.shape
    b = query.shape[0]
    cand = _sim_topk(
        query, episode_embeddings, W1, b1.reshape(1, -1), W2,
        b2.reshape(1, -1), drop_masks, chunk=1000)
    nck = cand.shape[0]
    cand = cand.transpose(1, 0, 2)
    cs = cand[:, :, 0:K].reshape(b, nck * K)
    cc = cand[:, :, K:2 * K].reshape(b, nck * K)
    cv = cand[:, :, 2 * K:3 * K].reshape(b, nck * K)
    scores, idx, uncert = _final_topk(cs, cc, cv)
    rows = _sc_gather(episodes, idx.reshape(-1))
    retrieved = rows.reshape(idx.shape[0], K, seq, h_dim)
    return retrieved, scores, uncert


# speedup vs baseline: 1.1401x; 1.1401x over previous
import functools

import jax
import jax.numpy as jnp
from jax import lax
from jax.experimental import pallas as pl
from jax.experimental.pallas import tpu as pltpu
from jax.experimental.pallas import tpu_sc as plsc

K = 8
NEG = float("-inf")
IMAX = 2**31 - 1


def _extract_topk(scores, ncol, vars_):
    s_list, c_list, v_list = [], [], []
    work = scores
    for _ in range(K):
        m = jnp.max(work, axis=1, keepdims=True)
        hit = work == m
        nc = jnp.max(jnp.where(hit, ncol, NEG), axis=1, keepdims=True)
        sel = ncol == nc
        v = jnp.max(jnp.where(sel, vars_, NEG), axis=1, keepdims=True)
        work = jnp.where(sel, NEG, work)
        s_list.append(m)
        c_list.append(nc)
        v_list.append(v)
    return (jnp.concatenate(s_list, axis=1),
            jnp.concatenate(c_list, axis=1),
            jnp.concatenate(v_list, axis=1))


def _sim_topk_body(q_ref, emb_ref, w1_ref, b1_ref, w2_ref, b2_ref, mask_ref,
                   s_out, c_out, v_out, *, chunk, nchunks):
    i = pl.program_id(0)
    b = q_ref.shape[0]
    mc = mask_ref.shape[0]

    q = q_ref[...]
    qn = q / jnp.maximum(jnp.sqrt(jnp.sum(q * q, axis=-1, keepdims=True)),
                         1e-8)
    emb = emb_ref[...]
    h = jnp.maximum(jnp.dot(emb, w1_ref[...]) + b1_ref[...], 0.0)
    ssum = None
    ssq = None
    for m_i in range(mc):
        noisy = jnp.dot(h * mask_ref[m_i], w2_ref[...]) + b2_ref[...]
        en = noisy / jnp.maximum(
            jnp.sqrt(jnp.sum(noisy * noisy, axis=-1, keepdims=True)), 1e-8)
        s = lax.dot_general(qn, en, (((1,), (1,)), ((), ())))
        ssum = s if ssum is None else ssum + s
        ssq = s * s if ssq is None else ssq + s * s
    mean = ssum / mc
    var = (ssq - mc * (mean * mean)) / (mc - 1)

    ncol = (-(lax.broadcasted_iota(jnp.int32, (b, chunk), 1)
              + i * chunk)).astype(jnp.float32)
    ch_s, ch_c, ch_v = _extract_topk(mean, ncol, var)

    s_out[0] = ch_s
    c_out[0] = ch_c
    v_out[0] = ch_v


def _sim_topk(query, emb, w1, b1, w2, b2, masks, *, chunk, interpret=False):
    b, h_dim = query.shape
    cap = emb.shape[0]
    mc = masks.shape[0]
    assert cap % chunk == 0
    nchunks = cap // chunk
    body = functools.partial(_sim_topk_body, chunk=chunk, nchunks=nchunks)
    return pl.pallas_call(
        body,
        grid=(nchunks,),
        in_specs=[
            pl.BlockSpec((b, h_dim), lambda i: (0, 0)),
            pl.BlockSpec((chunk, h_dim), lambda i: (i, 0)),
            pl.BlockSpec((h_dim, h_dim), lambda i: (0, 0)),
            pl.BlockSpec((1, h_dim), lambda i: (0, 0)),
            pl.BlockSpec((h_dim, h_dim), lambda i: (0, 0)),
            pl.BlockSpec((1, h_dim), lambda i: (0, 0)),
            pl.BlockSpec((mc, chunk, h_dim), lambda i: (0, i, 0)),
        ],
        out_specs=[
            pl.BlockSpec((1, b, K), lambda i: (i, 0, 0)),
            pl.BlockSpec((1, b, K), lambda i: (i, 0, 0)),
            pl.BlockSpec((1, b, K), lambda i: (i, 0, 0)),
        ],
        out_shape=[
            jax.ShapeDtypeStruct((nchunks, b, K), jnp.float32),
            jax.ShapeDtypeStruct((nchunks, b, K), jnp.float32),
            jax.ShapeDtypeStruct((nchunks, b, K), jnp.float32),
        ],
        interpret=interpret,
    )(query, emb, w1, b1, w2, b2, masks)


def _final_topk_body(s_ref, c_ref, v_ref, s_out, i_out, v_out):
    f_s, f_c, f_v = _extract_topk(s_ref[...], c_ref[...], v_ref[...])
    s_out[...] = f_s
    i_out[...] = (-f_c).astype(jnp.int32)
    v_out[...] = f_v


def _final_topk(cand_s, cand_c, cand_v, *, interpret=False):
    b = cand_s.shape[0]
    return pl.pallas_call(
        _final_topk_body,
        out_shape=[
            jax.ShapeDtypeStruct((b, K), jnp.float32),
            jax.ShapeDtypeStruct((b, K), jnp.int32),
            jax.ShapeDtypeStruct((b, K), jnp.float32),
        ],
        interpret=interpret,
    )(cand_s, cand_c, cand_v)


def _sc_gather(table, idx):
    n = idx.shape[0]
    _, seq, h_dim = table.shape
    info = plsc.get_sparse_core_info()
    nw = info.num_cores * info.num_subcores
    bpw = n // nw
    ch = 32
    assert bpw % ch == 0
    nch = bpw // ch
    mesh = plsc.VectorSubcoreMesh(core_axis_name="c", subcore_axis_name="s")

    @functools.partial(
        pl.kernel,
        out_type=jax.ShapeDtypeStruct((n, seq, h_dim), jnp.float32),
        mesh=mesh,
        scratch_types=[
            pltpu.VMEM((ch,), jnp.int32),
            pltpu.VMEM((ch, seq, h_dim), jnp.float32),
            pltpu.SemaphoreType.DMA,
        ],
    )
    def gather_kernel(table_hbm, idx_hbm, out_hbm, idx_v, rows_v, sem):
        wid = lax.axis_index("s") * info.num_cores + lax.axis_index("c")
        base = wid * bpw

        def step(c, carry):
            off = base + c * ch
            pltpu.sync_copy(idx_hbm.at[pl.ds(off, ch)], idx_v)
            pltpu.async_copy(table_hbm.at[idx_v], rows_v, sem).wait()
            pltpu.sync_copy(rows_v, out_hbm.at[pl.ds(off, ch)])
            return carry

        lax.fori_loop(0, nch, step, 0)

    return gather_kernel(table, idx)


def kernel(query, k, episodes, episode_embeddings, W1, b1, W2, b2, drop_masks):
    del k
    cap, seq, h_dim = episodes.shape
    b = query.shape[0]
    cs, cc, cv = _sim_topk(
        query, episode_embeddings, W1, b1.reshape(1, -1), W2,
        b2.reshape(1, -1), drop_masks, chunk=1000)
    nck = cs.shape[0]
    cs = cs.transpose(1, 0, 2).reshape(b, nck * K)
    cc = cc.transpose(1, 0, 2).reshape(b, nck * K)
    cv = cv.transpose(1, 0, 2).reshape(b, nck * K)
    scores, idx, uncert = _final_topk(cs, cc, cv)
    rows = _sc_gather(episodes, idx.reshape(-1))
    retrieved = rows.reshape(idx.shape[0], K, seq, h_dim)
    return retrieved, scores, uncert


# speedup vs baseline: 1.1417x; 1.0014x over previous
import functools

import jax
import jax.numpy as jnp
from jax import lax
from jax.experimental import pallas as pl
from jax.experimental.pallas import tpu as pltpu
from jax.experimental.pallas import tpu_sc as plsc

K = 8
NEG = float("-inf")


def _extract_topk(scores, ncol, vars_):
    s_list, c_list, v_list = [], [], []
    work = scores
    for _ in range(K):
        m = jnp.max(work, axis=1, keepdims=True)
        hit = work == m
        nc = jnp.max(jnp.where(hit, ncol, NEG), axis=1, keepdims=True)
        sel = ncol == nc
        v = jnp.max(jnp.where(sel, vars_, NEG), axis=1, keepdims=True)
        work = jnp.where(sel, NEG, work)
        s_list.append(m)
        c_list.append(nc)
        v_list.append(v)
    return (jnp.concatenate(s_list, axis=1),
            jnp.concatenate(c_list, axis=1),
            jnp.concatenate(v_list, axis=1))


def _sim_topk_body(q_ref, emb_ref, w1_ref, b1_ref, w2_ref, b2_ref, mask_ref,
                   s_out, c_out, v_out, *, chunk, nchunks):
    i = pl.program_id(0)
    b = q_ref.shape[0]
    mc = mask_ref.shape[0]

    q = q_ref[...]
    qn = q / jnp.maximum(jnp.sqrt(jnp.sum(q * q, axis=-1, keepdims=True)),
                         1e-8)
    emb = emb_ref[...]
    h = jnp.maximum(jnp.dot(emb, w1_ref[...]) + b1_ref[...], 0.0)
    ssum = None
    ssq = None
    for m_i in range(mc):
        noisy = jnp.dot(h * mask_ref[m_i], w2_ref[...]) + b2_ref[...]
        en = noisy / jnp.maximum(
            jnp.sqrt(jnp.sum(noisy * noisy, axis=-1, keepdims=True)), 1e-8)
        s = lax.dot_general(qn, en, (((1,), (1,)), ((), ())))
        ssum = s if ssum is None else ssum + s
        ssq = s * s if ssq is None else ssq + s * s
    mean = ssum / mc
    var = (ssq - mc * (mean * mean)) / (mc - 1)

    ncol = (-(lax.broadcasted_iota(jnp.int32, (b, chunk), 1)
              + i * chunk)).astype(jnp.float32)
    ch_s, ch_c, ch_v = _extract_topk(mean, ncol, var)

    s_out[0] = ch_s
    c_out[0] = ch_c
    v_out[0] = ch_v


def _sim_topk(query, emb, w1, b1, w2, b2, masks, *, chunk, interpret=False):
    b, h_dim = query.shape
    cap = emb.shape[0]
    mc = masks.shape[0]
    assert cap % chunk == 0
    nchunks = cap // chunk
    body = functools.partial(_sim_topk_body, chunk=chunk, nchunks=nchunks)
    return pl.pallas_call(
        body,
        grid=(nchunks,),
        in_specs=[
            pl.BlockSpec((b, h_dim), lambda i: (0, 0)),
            pl.BlockSpec((chunk, h_dim), lambda i: (i, 0)),
            pl.BlockSpec((h_dim, h_dim), lambda i: (0, 0)),
            pl.BlockSpec((1, h_dim), lambda i: (0, 0)),
            pl.BlockSpec((h_dim, h_dim), lambda i: (0, 0)),
            pl.BlockSpec((1, h_dim), lambda i: (0, 0)),
            pl.BlockSpec((mc, chunk, h_dim), lambda i: (0, i, 0)),
        ],
        out_specs=[
            pl.BlockSpec((1, b, K), lambda i: (i, 0, 0)),
            pl.BlockSpec((1, b, K), lambda i: (i, 0, 0)),
            pl.BlockSpec((1, b, K), lambda i: (i, 0, 0)),
        ],
        out_shape=[
            jax.ShapeDtypeStruct((nchunks, b, K), jnp.float32),
            jax.ShapeDtypeStruct((nchunks, b, K), jnp.float32),
            jax.ShapeDtypeStruct((nchunks, b, K), jnp.float32),
        ],
        interpret=interpret,
    )(query, emb, w1, b1, w2, b2, masks)


def _final_topk_body(s_ref, c_ref, v_ref, s_out, i_out, v_out):
    f_s, f_c, f_v = _extract_topk(s_ref[...], c_ref[...], v_ref[...])
    s_out[...] = f_s
    i_out[...] = (-f_c).astype(jnp.int32)
    v_out[...] = f_v


def _final_topk(cand_s, cand_c, cand_v, *, interpret=False):
    b = cand_s.shape[0]
    return pl.pallas_call(
        _final_topk_body,
        out_shape=[
            jax.ShapeDtypeStruct((b, K), jnp.float32),
            jax.ShapeDtypeStruct((b, K), jnp.int32),
            jax.ShapeDtypeStruct((b, K), jnp.float32),
        ],
        interpret=interpret,
    )(cand_s, cand_c, cand_v)


def _sc_gather(table, idx):
    n = idx.shape[0]
    _, seq, h_dim = table.shape
    info = plsc.get_sparse_core_info()
    nw = info.num_cores * info.num_subcores
    bpw = n // nw
    ch = 32
    assert bpw % ch == 0
    nch = bpw // ch
    mesh = plsc.VectorSubcoreMesh(core_axis_name="c", subcore_axis_name="s")

    @functools.partial(
        pl.kernel,
        out_type=jax.ShapeDtypeStruct((n, seq, h_dim), jnp.float32),
        mesh=mesh,
        scratch_types=[
            pltpu.VMEM((ch,), jnp.int32),
            pltpu.VMEM((ch, seq, h_dim), jnp.float32),
            pltpu.SemaphoreType.DMA,
        ],
    )
    def gather_kernel(table_hbm, idx_hbm, out_hbm, idx_v, rows_v, sem):
        wid = lax.axis_index("s") * info.num_cores + lax.axis_index("c")
        base = wid * bpw

        def step(c, carry):
            off = base + c * ch
            pltpu.sync_copy(idx_hbm.at[pl.ds(off, ch)], idx_v)
            pltpu.async_copy(table_hbm.at[idx_v], rows_v, sem).wait()
            pltpu.sync_copy(rows_v, out_hbm.at[pl.ds(off, ch)])
            return carry

        lax.fori_loop(0, nch, step, 0)

    return gather_kernel(table, idx)


def kernel(query, k, episodes, episode_embeddings, W1, b1, W2, b2, drop_masks):
    del k
    cap, seq, h_dim = episodes.shape
    b = query.shape[0]
    cs, cc, cv = _sim_topk(
        query, episode_embeddings, W1, b1.reshape(1, -1), W2,
        b2.reshape(1, -1), drop_masks, chunk=1000)
    nck = cs.shape[0]
    cs = cs.transpose(1, 0, 2).reshape(b, nck * K)
    cc = cc.transpose(1, 0, 2).reshape(b, nck * K)
    cv = cv.transpose(1, 0, 2).reshape(b, nck * K)
    scores, idx, uncert = _final_topk(cs, cc, cv)
    rows = _sc_gather(episodes, idx.reshape(-1))
    retrieved = rows.reshape(idx.shape[0], K, seq, h_dim)
    return retrieved, scores, uncert
